# baseline (speedup 1.0000x reference)
import jax
import jax.numpy as jnp
from jax import lax
from jax.experimental import pallas as pl
from jax.experimental.pallas import tpu as pltpu
from jax.experimental.pallas import tpu_sc as plsc

L = 20
B = 16384
N_STATES = 2 ** L

_NC = 2
_NS = 16
_NW = _NC * _NS
_BPW = B // _NW
_NCHUNK = _BPW // 128

_TC_GRID = 8
_TC_ROWS = B // _TC_GRID

_WEIGHTS = [2.0 ** (L - 2 - l) for l in range(L)]
_BIAS = 2.0 ** (L - 1) - 0.5


def _tc_idx_body(x_ref, idx_ref):
    x = x_ref[...]
    bit = jnp.where(x > 0.0, 1, 0).astype(jnp.int32)
    shift = (L - 1) - lax.broadcasted_iota(jnp.int32, (_TC_ROWS, L), 1)
    s = jnp.sum(lax.shift_left(bit, shift), axis=1)
    idx_ref[...] = s.reshape(_TC_ROWS // 128, 128)


def _sc_gather_body(idx_hbm, table_hbm, out_hbm, idxv, outv, sem):
    wid = lax.axis_index("s") * _NC + lax.axis_index("c")
    base = wid * _BPW

    pltpu.sync_copy(idx_hbm.at[pl.ds(wid * _NCHUNK, _NCHUNK), :], idxv)

    copies = [
        pltpu.async_copy(
            table_hbm.at[idxv.at[c]], outv.at[pl.ds(c * 128, 128)], sem
        )
        for c in range(_NCHUNK)
    ]
    for cp in copies:
        cp.wait()
    pltpu.sync_copy(outv, out_hbm.at[pl.ds(base, _BPW)])


@jax.jit
def kernel(x_in, logstate):
    idx2d = pl.pallas_call(
        _tc_idx_body,
        grid=(_TC_GRID,),
        in_specs=[pl.BlockSpec((_TC_ROWS, L), lambda i: (i, 0))],
        out_specs=pl.BlockSpec((_TC_ROWS // 128, 128), lambda i: (i, 0)),
        out_shape=jax.ShapeDtypeStruct((B // 128, 128), jnp.int32),
    )(x_in)

    mesh = plsc.VectorSubcoreMesh(core_axis_name="c", subcore_axis_name="s")
    run = pl.kernel(
        _sc_gather_body,
        mesh=mesh,
        out_type=jax.ShapeDtypeStruct((B,), jnp.float32),
        scratch_types=[
            pltpu.VMEM((_NCHUNK, 128), jnp.int32),
            pltpu.VMEM((_BPW,), jnp.float32),
            pltpu.SemaphoreType.DMA,
        ],
        compiler_params=pltpu.CompilerParams(needs_layout_passes=False),
    )
    return run(idx2d, logstate)


# speedup vs baseline: 1.5111x; 1.5111x over previous
import jax
import jax.numpy as jnp
from jax import lax
from jax.experimental import pallas as pl
from jax.experimental.pallas import tpu as pltpu
from jax.experimental.pallas import tpu_sc as plsc

L = 20
B = 16384
N_STATES = 2 ** L

_NC = 2
_NS = 16
_NW = _NC * _NS
_BPW = B // _NW
_NCHUNK = _BPW // 128


def _sc_body(xt_hbm, table_hbm, out_hbm, xtv, idxv, outv, sem):
    wid = lax.axis_index("s") * _NC + lax.axis_index("c")
    base = wid * _BPW

    pltpu.sync_copy(xt_hbm.at[:, pl.ds(base, _BPW)], xtv)

    bias = jnp.full((16,), 2.0 ** (L - 1) - 0.5, jnp.float32)

    def body(j, _):
        col = j * 16
        accs = [None, None, None, None]
        for l in range(L):
            v = xtv[l, pl.ds(col, 16)]
            term = v * jnp.float32(2.0 ** (L - 2 - l))
            a = l % 4
            accs[a] = term if accs[a] is None else accs[a] + term
        acc = (accs[0] + accs[1]) + (accs[2] + accs[3]) + bias
        idxv[j // 8, pl.ds((j % 8) * 16, 16)] = acc.astype(jnp.int32)
        return 0

    lax.fori_loop(0, 8 * _NCHUNK, body, 0)

    copies = [
        pltpu.async_copy(
            table_hbm.at[idxv.at[c]], outv.at[pl.ds(c * 128, 128)], sem
        )
        for c in range(_NCHUNK)
    ]
    for cp in copies:
        cp.wait()
    pltpu.sync_copy(outv, out_hbm.at[pl.ds(base, _BPW)])


@jax.jit
def kernel(x_in, logstate):
    mesh = plsc.VectorSubcoreMesh(core_axis_name="c", subcore_axis_name="s")
    run = pl.kernel(
        _sc_body,
        mesh=mesh,
        out_type=jax.ShapeDtypeStruct((B,), jnp.float32),
        scratch_types=[
            pltpu.VMEM((L, _BPW), jnp.float32),
            pltpu.VMEM((_NCHUNK, 128), jnp.int32),
            pltpu.VMEM((_BPW,), jnp.float32),
            pltpu.SemaphoreType.DMA,
        ],
        compiler_params=pltpu.CompilerParams(needs_layout_passes=False),
    )
    return run(x_in.T, logstate)


# speedup vs baseline: 1.5142x; 1.0021x over previous
import jax
import jax.numpy as jnp
from jax import lax
from jax.experimental import pallas as pl
from jax.experimental.pallas import tpu as pltpu
from jax.experimental.pallas import tpu_sc as plsc

L = 20
B = 16384
N_STATES = 2 ** L

_NC = 2
_NS = 16
_NW = _NC * _NS
_BPW = B // _NW
_NCHUNK = _BPW // 128


def _sc_body(xt_hbm, table_hbm, out_hbm, xtv, idxv, outv, sem, sga, sgb):
    wid = lax.axis_index("s") * _NC + lax.axis_index("c")
    base = wid * _BPW
    half = _BPW // 2

    stg_a = pltpu.async_copy(
        xt_hbm.at[:, pl.ds(base, half)], xtv.at[:, pl.ds(0, half)], sga
    )
    stg_b = pltpu.async_copy(
        xt_hbm.at[:, pl.ds(base + half, half)], xtv.at[:, pl.ds(half, half)], sgb
    )

    bias = jnp.full((16,), 2.0 ** (L - 1) - 0.5, jnp.float32)

    def body(j, _):
        col = j * 16
        accs = [None, None, None, None]
        for l in range(L):
            v = xtv[l, pl.ds(col, 16)]
            term = v * jnp.float32(2.0 ** (L - 2 - l))
            a = l % 4
            accs[a] = term if accs[a] is None else accs[a] + term
        acc = (accs[0] + accs[1]) + (accs[2] + accs[3]) + bias
        idxv[j // 8, pl.ds((j % 8) * 16, 16)] = acc.astype(jnp.int32)
        return 0

    copies = []
    stg_a.wait()
    lax.fori_loop(0, 16, body, 0)
    for c in (0, 1):
        copies.append(
            pltpu.async_copy(
                table_hbm.at[idxv.at[c]], outv.at[pl.ds(c * 128, 128)], sem
            )
        )
    stg_b.wait()
    lax.fori_loop(16, 32, body, 0)
    for c in (2, 3):
        copies.append(
            pltpu.async_copy(
                table_hbm.at[idxv.at[c]], outv.at[pl.ds(c * 128, 128)], sem
            )
        )
    for cp in copies:
        cp.wait()
    pltpu.sync_copy(outv, out_hbm.at[pl.ds(base, _BPW)])


@jax.jit
def kernel(x_in, logstate):
    mesh = plsc.VectorSubcoreMesh(core_axis_name="c", subcore_axis_name="s")
    run = pl.kernel(
        _sc_body,
        mesh=mesh,
        out_type=jax.ShapeDtypeStruct((B,), jnp.float32),
        scratch_types=[
            pltpu.VMEM((L, _BPW), jnp.float32),
            pltpu.VMEM((_NCHUNK, 128), jnp.int32),
            pltpu.VMEM((_BPW,), jnp.float32),
            pltpu.SemaphoreType.DMA,
            pltpu.SemaphoreType.DMA,
            pltpu.SemaphoreType.DMA,
        ],
        compiler_params=pltpu.CompilerParams(needs_layout_passes=False),
    )
    return run(x_in.T, logstate)
